# baseline (speedup 1.0000x reference)
import jax
import jax.numpy as jnp
from jax import lax
from jax.experimental import pallas as pl
from jax.experimental.pallas import tpu as pltpu
from jax.experimental.pallas import tpu_sc as plsc

N = 10000
E = 320000
D = 128
NC, NS = 2, 16
NW = NC * NS
CH = 64
NCPQ = 40
EPQ = NCPQ * CH
QT = 8
NQ0, NQ1 = 4, 4
NQTOT = NS * QT
E_PAD = NQTOT * EPQ
APAD = 10240
STR = APAD // NS

_GDN = lax.GatherDimensionNumbers(
    offset_dims=(), collapsed_slice_dims=(0,), start_index_map=(0,))


def _bcast16(v, lane):
  idx = jnp.full((16,), lane, jnp.int32)
  return lax.gather(v, idx[:, None], _GDN, (1,),
                    mode=lax.GatherScatterMode.PROMISE_IN_BOUNDS)


def _zero_vmem_1d(ref, n):
  def body(i, _):
    ref[pl.ds(i * 16, 16)] = jnp.zeros((16,), jnp.float32)
    return 0
  lax.fori_loop(0, n // 16, body, 0)


def _zero_vmem_rows(ref, rows):
  def body(i, _):
    for j in range(D // 16):
      ref[i, pl.ds(j * 16, 16)] = jnp.zeros((16,), jnp.float32)
    return 0
  lax.fori_loop(0, rows, body, 0)


def _worker_quarters(c, s):
  qbase = jnp.where(c == 0, s * NQ0, NS * NQ0 + s * NQ1)
  nq = jnp.where(c == 0, NQ0, NQ1)
  return qbase, nq


def _deg_body(dst3_hbm, ew3_hbm, out_hbm, dstb, ewb, z_v, acc_sh, dsem):
  c = lax.axis_index("c")
  s = lax.axis_index("s")
  qbase, nq = _worker_quarters(c, s)

  _zero_vmem_1d(z_v, STR)
  pltpu.sync_copy(z_v, acc_sh.at[pl.ds(s * STR, STR)])
  plsc.subcore_barrier()

  def quarter(q, _):
    pltpu.sync_copy(dst3_hbm.at[qbase + q], dstb)
    pltpu.sync_copy(ew3_hbm.at[qbase + q], ewb)

    def chunk(i, _):
      pltpu.async_copy(ewb.at[i], acc_sh.at[dstb.at[i]], dsem, add=True)

      @pl.when(i >= 16)
      def _():
        pltpu.make_async_copy(ewb.at[0], acc_sh.at[pl.ds(0, CH)],
                              dsem).wait()
      return 0
    lax.fori_loop(0, NCPQ, chunk, 0)
    for _ in range(16):
      pltpu.make_async_copy(ewb.at[0], acc_sh.at[pl.ds(0, CH)],
                            dsem).wait()
    return 0
  lax.fori_loop(0, nq, quarter, 0)

  plsc.subcore_barrier()
  pltpu.sync_copy(acc_sh.at[pl.ds(s * STR, STR)],
                  out_hbm.at[pl.ds(c * APAD + s * STR, STR)])


_deg_call = pl.kernel(
    _deg_body,
    out_type=jax.ShapeDtypeStruct((NC * APAD,), jnp.float32),
    mesh=plsc.VectorSubcoreMesh(core_axis_name="c", subcore_axis_name="s"),
    scratch_types=[
        pltpu.VMEM((NCPQ, CH), jnp.int32),
        pltpu.VMEM((NCPQ, CH), jnp.float32),
        pltpu.VMEM((STR,), jnp.float32),
        pltpu.VMEM_SHARED((APAD,), jnp.float32),
        pltpu.SemaphoreType.DMA,
    ],
)


def _agg_body(hp_hbm, src3_hbm, dst3_hbm, ew3_hbm, out_hbm,
              srcb, dstb, ewb, r0, r1, r2, r3,
              g0, g1, g2, g3, s0, s1, s2, s3, acc_sh):
  c = lax.axis_index("c")
  s = lax.axis_index("s")
  qbase, nq = _worker_quarters(c, s)
  rows = (r0, r1, r2, r3)
  gsem = (g0, g1, g2, g3)
  ssem = (s0, s1, s2, s3)

  with jax.named_scope("agg_zero"):
    _zero_vmem_rows(r0, CH)
    for t in range(STR // CH):
      pltpu.sync_copy(r0, acc_sh.at[pl.ds(s * STR + t * CH, CH)])
    plsc.subcore_barrier()

  def _scale(i, b):
    def grp(g, _):
      wv = ewb[i, pl.ds(g * 16, 16)]
      for e in range(16):
        wsp = _bcast16(wv, e)
        r = g * 16 + e
        for j in range(D // 16):
          rows[b][r, pl.ds(j * 16, 16)] = rows[b][r, pl.ds(j * 16, 16)] * wsp
      return 0
    lax.fori_loop(0, CH // 16, grp, 0)

  def _step(i, b):
    pltpu.make_async_copy(hp_hbm.at[pl.ds(0, CH)], rows[b], gsem[b]).wait()
    _scale(i, b)
    pltpu.async_copy(rows[b], acc_sh.at[dstb.at[i]], ssem[b], add=True)
    b1 = (b + 1) % 4

    @pl.when(jnp.logical_and(i >= 3, i + 1 < NCPQ))
    def _():
      pltpu.make_async_copy(rows[b1], acc_sh.at[pl.ds(0, CH)],
                            ssem[b1]).wait()

    @pl.when(i + 1 < NCPQ)
    def _():
      pltpu.async_copy(hp_hbm.at[srcb.at[i + 1]], rows[b1], gsem[b1])

  def quarter(q, _):
    pltpu.sync_copy(src3_hbm.at[qbase + q], srcb)
    pltpu.sync_copy(dst3_hbm.at[qbase + q], dstb)
    pltpu.sync_copy(ew3_hbm.at[qbase + q], ewb)
    pltpu.async_copy(hp_hbm.at[srcb.at[0]], rows[0], gsem[0])

    def quad(j, _):
      for t in range(4):
        _step(j * 4 + t, t)
      return 0
    lax.fori_loop(0, NCPQ // 4, quad, 0)
    for b in range(4):
      pltpu.make_async_copy(rows[b], acc_sh.at[pl.ds(0, CH)], ssem[b]).wait()
    return 0
  with jax.named_scope("agg_pipeline"):
    lax.fori_loop(0, nq, quarter, 0)

  with jax.named_scope("agg_tail"):
    plsc.subcore_barrier()
    pltpu.sync_copy(acc_sh.at[pl.ds(s * STR, STR)],
                    out_hbm.at[c, pl.ds(s * STR, STR)])


_agg_call = pl.kernel(
    _agg_body,
    out_type=jax.ShapeDtypeStruct((NC, APAD, D), jnp.float32),
    mesh=plsc.VectorSubcoreMesh(core_axis_name="c", subcore_axis_name="s"),
    scratch_types=[
        pltpu.VMEM((NCPQ, CH), jnp.int32),
        pltpu.VMEM((NCPQ, CH), jnp.int32),
        pltpu.VMEM((NCPQ, CH), jnp.float32),
        pltpu.VMEM((CH, D), jnp.float32),
        pltpu.VMEM((CH, D), jnp.float32),
        pltpu.VMEM((CH, D), jnp.float32),
        pltpu.VMEM((CH, D), jnp.float32),
        pltpu.SemaphoreType.DMA,
        pltpu.SemaphoreType.DMA,
        pltpu.SemaphoreType.DMA,
        pltpu.SemaphoreType.DMA,
        pltpu.SemaphoreType.DMA,
        pltpu.SemaphoreType.DMA,
        pltpu.SemaphoreType.DMA,
        pltpu.SemaphoreType.DMA,
        pltpu.VMEM_SHARED((APAD, D), jnp.float32),
    ],
)


def _leaky(x):
  return jnp.where(x >= 0.0, x, 0.1 * x)


def _bn(x, gamma, beta):
  mu = jnp.mean(x, axis=0, keepdims=True)
  xc = x - mu
  var = jnp.mean(xc * xc, axis=0, keepdims=True)
  return gamma * xc * lax.rsqrt(var + 1e-5) + beta


def _tc1_body(x_ref, d0_ref, d1_ref, g_ref, bt_ref, w_ref, h1p_ref, dinv_ref):
  x = x_ref[...]
  xa = _leaky(_bn(x, g_ref[...], bt_ref[...]))
  deg = d0_ref[...] + d1_ref[...] + 1.0
  dinv = jnp.where(deg > 0.0, lax.rsqrt(deg), 0.0)
  h = jnp.dot(xa, w_ref[...], preferred_element_type=jnp.float32)
  h1p_ref[...] = dinv * h
  dinv_ref[...] = dinv


def _tc2_body(p0_ref, p1_ref, hp_ref, dinv_ref, b1_ref, g_ref, bt_ref, w_ref,
              h2p_ref):
  dinv = dinv_ref[...]
  o1 = dinv * (p0_ref[...] + p1_ref[...] + hp_ref[...]) + b1_ref[...]
  a = _leaky(_bn(_leaky(o1), g_ref[...], bt_ref[...]))
  h2 = jnp.dot(a, w_ref[...], preferred_element_type=jnp.float32)
  h2p_ref[...] = dinv * h2


def _tc3_body(q0_ref, q1_ref, hp_ref, dinv_ref, b2_ref, out_ref):
  o = dinv_ref[...] * (q0_ref[...] + q1_ref[...] + hp_ref[...]) + b2_ref[...]
  out_ref[...] = _leaky(o)


_f32 = jnp.float32
_tc1_call = pl.pallas_call(
    _tc1_body,
    out_shape=(jax.ShapeDtypeStruct((N, D), _f32),
               jax.ShapeDtypeStruct((N, 1), _f32)),
)
_tc2_call = pl.pallas_call(
    _tc2_body,
    out_shape=jax.ShapeDtypeStruct((N, D), _f32),
)
_tc3_call = pl.pallas_call(
    _tc3_body,
    out_shape=jax.ShapeDtypeStruct((N, D), _f32),
)


def kernel(x, edge_index, edge_attr, bn1_gamma, bn1_beta, W1, b1,
           bn2_gamma, bn2_beta, W2, b2):
  src = edge_index[0]
  dst = edge_index[1]
  ew = edge_attr[:, 0]

  pad = E_PAD - E
  pi = jnp.arange(pad, dtype=jnp.int32) % N
  src3 = jnp.concatenate([src, pi]).reshape(NQTOT, NCPQ, CH)
  dst3 = jnp.concatenate([dst, pi]).reshape(NQTOT, NCPQ, CH)
  ew3 = jnp.concatenate([ew, jnp.zeros((pad,), jnp.float32)]
                        ).reshape(NQTOT, NCPQ, CH)

  deg_parts = _deg_call(dst3, ew3)
  d0 = deg_parts[:N, None]
  d1 = deg_parts[APAD:APAD + N, None]

  h1p, dinv = _tc1_call(x, d0, d1, bn1_gamma[None, :], bn1_beta[None, :], W1)

  p = _agg_call(h1p, src3, dst3, ew3)
  h2p = _tc2_call(p[0, :N], p[1, :N], h1p, dinv, b1[None, :],
                  bn2_gamma[None, :], bn2_beta[None, :], W2)

  q = _agg_call(h2p, src3, dst3, ew3)
  out = _tc3_call(q[0, :N], q[1, :N], h2p, dinv, b2[None, :])
  return (out, edge_index)


# speedup vs baseline: 1.4338x; 1.4338x over previous
import jax
import jax.numpy as jnp
from jax import lax
from jax.experimental import pallas as pl
from jax.experimental.pallas import tpu as pltpu
from jax.experimental.pallas import tpu_sc as plsc

N = 10000
E = 320000
D = 128
NC, NS = 2, 16
NW = NC * NS
CH = 32
NCPQ = 40
EPQ = NCPQ * CH
NQ0, NQ1 = 8, 8
NQTOT = NS * (NQ0 + NQ1)
E_PAD = NQTOT * EPQ
APAD = 10240
STR = APAD // NS

_GDN = lax.GatherDimensionNumbers(
    offset_dims=(), collapsed_slice_dims=(0,), start_index_map=(0,))


def _bcast16(v, lane):
  idx = jnp.full((16,), lane, jnp.int32)
  return lax.gather(v, idx[:, None], _GDN, (1,),
                    mode=lax.GatherScatterMode.PROMISE_IN_BOUNDS)


def _zero_vmem_1d(ref, n):
  def body(i, _):
    ref[pl.ds(i * 16, 16)] = jnp.zeros((16,), jnp.float32)
    return 0
  lax.fori_loop(0, n // 16, body, 0)


def _zero_vmem_rows(ref, rows):
  def body(i, _):
    for j in range(D // 16):
      ref[i, pl.ds(j * 16, 16)] = jnp.zeros((16,), jnp.float32)
    return 0
  lax.fori_loop(0, rows, body, 0)


def _worker_quarters(c, s):
  qbase = jnp.where(c == 0, s * NQ0, NS * NQ0 + s * NQ1)
  nq = jnp.where(c == 0, NQ0, NQ1)
  return qbase, nq


def _deg_body(dst3_hbm, ew3_hbm, out_hbm, dstb, ewb, z_v, acc_sh, dsem):
  c = lax.axis_index("c")
  s = lax.axis_index("s")
  qbase, nq = _worker_quarters(c, s)

  _zero_vmem_1d(z_v, STR)
  pltpu.sync_copy(z_v, acc_sh.at[pl.ds(s * STR, STR)])
  plsc.subcore_barrier()

  def quarter(q, _):
    pltpu.sync_copy(dst3_hbm.at[qbase + q], dstb)
    pltpu.sync_copy(ew3_hbm.at[qbase + q], ewb)

    def chunk(i, _):
      pltpu.async_copy(ewb.at[i], acc_sh.at[dstb.at[i]], dsem, add=True)

      @pl.when(i >= 16)
      def _():
        pltpu.make_async_copy(ewb.at[0], acc_sh.at[pl.ds(0, CH)],
                              dsem).wait()
      return 0
    lax.fori_loop(0, NCPQ, chunk, 0)
    for _ in range(16):
      pltpu.make_async_copy(ewb.at[0], acc_sh.at[pl.ds(0, CH)],
                            dsem).wait()
    return 0
  lax.fori_loop(0, nq, quarter, 0)

  plsc.subcore_barrier()
  pltpu.sync_copy(acc_sh.at[pl.ds(s * STR, STR)],
                  out_hbm.at[pl.ds(c * APAD + s * STR, STR)])


_deg_call = pl.kernel(
    _deg_body,
    out_type=jax.ShapeDtypeStruct((NC * APAD,), jnp.float32),
    mesh=plsc.VectorSubcoreMesh(core_axis_name="c", subcore_axis_name="s"),
    scratch_types=[
        pltpu.VMEM((NCPQ, CH), jnp.int32),
        pltpu.VMEM((NCPQ, CH), jnp.float32),
        pltpu.VMEM((STR,), jnp.float32),
        pltpu.VMEM_SHARED((APAD,), jnp.float32),
        pltpu.SemaphoreType.DMA,
    ],
)


NBUF = 8
GLEAD = 4


def _agg_body(hp_hbm, src3_hbm, dst3_hbm, ew3_hbm, out_hbm,
              srcb, dstb, ewb, r0, r1, r2, r3, r4, r5, r6, r7,
              g0, g1, g2, g3, g4, g5, g6, g7,
              s0, s1, s2, s3, s4, s5, s6, s7, acc_sh):
  c = lax.axis_index("c")
  s = lax.axis_index("s")
  qbase, nq = _worker_quarters(c, s)
  rows = (r0, r1, r2, r3, r4, r5, r6, r7)
  gsem = (g0, g1, g2, g3, g4, g5, g6, g7)
  ssem = (s0, s1, s2, s3, s4, s5, s6, s7)

  with jax.named_scope("agg_zero"):
    _zero_vmem_rows(r0, CH)
    for t in range(STR // CH):
      pltpu.sync_copy(r0, acc_sh.at[pl.ds(s * STR + t * CH, CH)])
    plsc.subcore_barrier()

  def _scale(i, b):
    def grp(g, _):
      wv = ewb[i, pl.ds(g * 16, 16)]
      for e in range(16):
        wsp = _bcast16(wv, e)
        r = g * 16 + e
        for j in range(D // 16):
          rows[b][r, pl.ds(j * 16, 16)] = rows[b][r, pl.ds(j * 16, 16)] * wsp
      return 0
    lax.fori_loop(0, CH // 16, grp, 0)

  def _step(i, b):
    pltpu.make_async_copy(hp_hbm.at[pl.ds(0, CH)], rows[b], gsem[b]).wait()
    _scale(i, b)
    pltpu.async_copy(rows[b], acc_sh.at[dstb.at[i]], ssem[b], add=True)
    bn = (b + GLEAD) % NBUF

    @pl.when(jnp.logical_and(i >= NBUF - GLEAD, i + GLEAD < NCPQ))
    def _():
      pltpu.make_async_copy(rows[bn], acc_sh.at[pl.ds(0, CH)],
                            ssem[bn]).wait()

    @pl.when(i + GLEAD < NCPQ)
    def _():
      pltpu.async_copy(hp_hbm.at[srcb.at[i + GLEAD]], rows[bn], gsem[bn])

  def quarter(q, _):
    pltpu.sync_copy(src3_hbm.at[qbase + q], srcb)
    pltpu.sync_copy(dst3_hbm.at[qbase + q], dstb)
    pltpu.sync_copy(ew3_hbm.at[qbase + q], ewb)
    for b in range(GLEAD):
      pltpu.async_copy(hp_hbm.at[srcb.at[b]], rows[b], gsem[b])

    def ring(j, _):
      for t in range(NBUF):
        _step(j * NBUF + t, t)
      return 0
    lax.fori_loop(0, NCPQ // NBUF, ring, 0)
    for b in range(NBUF):
      pltpu.make_async_copy(rows[b], acc_sh.at[pl.ds(0, CH)], ssem[b]).wait()
    return 0
  with jax.named_scope("agg_pipeline"):
    lax.fori_loop(0, nq, quarter, 0)

  with jax.named_scope("agg_tail"):
    plsc.subcore_barrier()
    pltpu.sync_copy(acc_sh.at[pl.ds(s * STR, STR)],
                    out_hbm.at[c, pl.ds(s * STR, STR)])


_agg_call = pl.kernel(
    _agg_body,
    out_type=jax.ShapeDtypeStruct((NC, APAD, D), jnp.float32),
    mesh=plsc.VectorSubcoreMesh(core_axis_name="c", subcore_axis_name="s"),
    scratch_types=[
        pltpu.VMEM((NCPQ, CH), jnp.int32),
        pltpu.VMEM((NCPQ, CH), jnp.int32),
        pltpu.VMEM((NCPQ, CH), jnp.float32),
    ] + [pltpu.VMEM((CH, D), jnp.float32)] * NBUF
      + [pltpu.SemaphoreType.DMA] * (2 * NBUF)
      + [pltpu.VMEM_SHARED((APAD, D), jnp.float32)],
)


def _leaky(x):
  return jnp.where(x >= 0.0, x, 0.1 * x)


def _bn(x, gamma, beta):
  mu = jnp.mean(x, axis=0, keepdims=True)
  xc = x - mu
  var = jnp.mean(xc * xc, axis=0, keepdims=True)
  return gamma * xc * lax.rsqrt(var + 1e-5) + beta


def _tc1_body(x_ref, d0_ref, d1_ref, g_ref, bt_ref, w_ref, h1p_ref, dinv_ref):
  x = x_ref[...]
  xa = _leaky(_bn(x, g_ref[...], bt_ref[...]))
  deg = d0_ref[...] + d1_ref[...] + 1.0
  dinv = jnp.where(deg > 0.0, lax.rsqrt(deg), 0.0)
  h = jnp.dot(xa, w_ref[...], preferred_element_type=jnp.float32)
  h1p_ref[...] = dinv * h
  dinv_ref[...] = dinv


def _tc2_body(p0_ref, p1_ref, hp_ref, dinv_ref, b1_ref, g_ref, bt_ref, w_ref,
              h2p_ref):
  dinv = dinv_ref[...]
  o1 = dinv * (p0_ref[...] + p1_ref[...] + hp_ref[...]) + b1_ref[...]
  a = _leaky(_bn(_leaky(o1), g_ref[...], bt_ref[...]))
  h2 = jnp.dot(a, w_ref[...], preferred_element_type=jnp.float32)
  h2p_ref[...] = dinv * h2


def _tc3_body(q0_ref, q1_ref, hp_ref, dinv_ref, b2_ref, out_ref):
  o = dinv_ref[...] * (q0_ref[...] + q1_ref[...] + hp_ref[...]) + b2_ref[...]
  out_ref[...] = _leaky(o)


_f32 = jnp.float32
_tc1_call = pl.pallas_call(
    _tc1_body,
    out_shape=(jax.ShapeDtypeStruct((N, D), _f32),
               jax.ShapeDtypeStruct((N, 1), _f32)),
)
_tc2_call = pl.pallas_call(
    _tc2_body,
    out_shape=jax.ShapeDtypeStruct((N, D), _f32),
)
_tc3_call = pl.pallas_call(
    _tc3_body,
    out_shape=jax.ShapeDtypeStruct((N, D), _f32),
)


def kernel(x, edge_index, edge_attr, bn1_gamma, bn1_beta, W1, b1,
           bn2_gamma, bn2_beta, W2, b2):
  src = edge_index[0]
  dst = edge_index[1]
  ew = edge_attr[:, 0]

  pad = E_PAD - E
  pi = jnp.arange(pad, dtype=jnp.int32) % N
  src3 = jnp.concatenate([src, pi]).reshape(NQTOT, NCPQ, CH)
  dst3 = jnp.concatenate([dst, pi]).reshape(NQTOT, NCPQ, CH)
  ew3 = jnp.concatenate([ew, jnp.zeros((pad,), jnp.float32)]
                        ).reshape(NQTOT, NCPQ, CH)

  deg_parts = _deg_call(dst3, ew3)
  d0 = deg_parts[:N, None]
  d1 = deg_parts[APAD:APAD + N, None]

  h1p, dinv = _tc1_call(x, d0, d1, bn1_gamma[None, :], bn1_beta[None, :], W1)

  p = _agg_call(h1p, src3, dst3, ew3)
  h2p = _tc2_call(p[0, :N], p[1, :N], h1p, dinv, b1[None, :],
                  bn2_gamma[None, :], bn2_beta[None, :], W2)

  q = _agg_call(h2p, src3, dst3, ew3)
  out = _tc3_call(q[0, :N], q[1, :N], h2p, dinv, b2[None, :])
  return (out, edge_index)


# speedup vs baseline: 1.4610x; 1.0190x over previous
import jax
import jax.numpy as jnp
from jax import lax
from jax.experimental import pallas as pl
from jax.experimental.pallas import tpu as pltpu
from jax.experimental.pallas import tpu_sc as plsc

N = 10000
E = 320000
D = 128
NC, NS = 2, 16
NW = NC * NS
CH = 64
NCPQ = 40
EPQ = NCPQ * CH
NQ0, NQ1 = 4, 4
NQTOT = NS * (NQ0 + NQ1)
E_PAD = NQTOT * EPQ
APAD = 10240
STR = APAD // NS

_GDN = lax.GatherDimensionNumbers(
    offset_dims=(), collapsed_slice_dims=(0,), start_index_map=(0,))


def _bcast16(v, lane):
  idx = jnp.full((16,), lane, jnp.int32)
  return lax.gather(v, idx[:, None], _GDN, (1,),
                    mode=lax.GatherScatterMode.PROMISE_IN_BOUNDS)


def _zero_vmem_1d(ref, n):
  def body(i, _):
    ref[pl.ds(i * 16, 16)] = jnp.zeros((16,), jnp.float32)
    return 0
  lax.fori_loop(0, n // 16, body, 0)


def _zero_vmem_rows(ref, rows):
  def body(i, _):
    for j in range(D // 16):
      ref[i, pl.ds(j * 16, 16)] = jnp.zeros((16,), jnp.float32)
    return 0
  lax.fori_loop(0, rows, body, 0)


def _worker_quarters(c, s):
  qbase = jnp.where(c == 0, s * NQ0, NS * NQ0 + s * NQ1)
  nq = jnp.where(c == 0, NQ0, NQ1)
  return qbase, nq


def _deg_body(dst3_hbm, ew3_hbm, out_hbm, dstb, ewb, z_v, acc_sh, dsem):
  c = lax.axis_index("c")
  s = lax.axis_index("s")
  qbase, nq = _worker_quarters(c, s)

  _zero_vmem_1d(z_v, STR)
  pltpu.sync_copy(z_v, acc_sh.at[pl.ds(s * STR, STR)])
  plsc.subcore_barrier()

  def quarter(q, _):
    pltpu.sync_copy(dst3_hbm.at[qbase + q], dstb)
    pltpu.sync_copy(ew3_hbm.at[qbase + q], ewb)

    def chunk(i, _):
      pltpu.async_copy(ewb.at[i], acc_sh.at[dstb.at[i]], dsem, add=True)

      @pl.when(i >= 16)
      def _():
        pltpu.make_async_copy(ewb.at[0], acc_sh.at[pl.ds(0, CH)],
                              dsem).wait()
      return 0
    lax.fori_loop(0, NCPQ, chunk, 0)
    for _ in range(16):
      pltpu.make_async_copy(ewb.at[0], acc_sh.at[pl.ds(0, CH)],
                            dsem).wait()
    return 0
  lax.fori_loop(0, nq, quarter, 0)

  plsc.subcore_barrier()
  pltpu.sync_copy(acc_sh.at[pl.ds(s * STR, STR)],
                  out_hbm.at[pl.ds(c * APAD + s * STR, STR)])


_deg_call = pl.kernel(
    _deg_body,
    out_type=jax.ShapeDtypeStruct((NC * APAD,), jnp.float32),
    mesh=plsc.VectorSubcoreMesh(core_axis_name="c", subcore_axis_name="s"),
    scratch_types=[
        pltpu.VMEM((NCPQ, CH), jnp.int32),
        pltpu.VMEM((NCPQ, CH), jnp.float32),
        pltpu.VMEM((STR,), jnp.float32),
        pltpu.VMEM_SHARED((APAD,), jnp.float32),
        pltpu.SemaphoreType.DMA,
    ],
)


NBUF = 4
GLEAD = 2


def _agg_body(hp_hbm, src3_hbm, dst3_hbm, ew3_hbm, out_hbm,
              srcb, dstb, ewb, r0, r1, r2, r3,
              g0, g1, g2, g3, s0, s1, s2, s3, acc_sh):
  c = lax.axis_index("c")
  s = lax.axis_index("s")
  qbase, nq = _worker_quarters(c, s)
  rows = (r0, r1, r2, r3)
  gsem = (g0, g1, g2, g3)
  ssem = (s0, s1, s2, s3)

  with jax.named_scope("agg_zero"):
    _zero_vmem_rows(r0, CH)
    for t in range(STR // CH):
      pltpu.sync_copy(r0, acc_sh.at[pl.ds(s * STR + t * CH, CH)])
    plsc.subcore_barrier()

  def _scale(i, b):
    def grp(g, _):
      wv = ewb[i, pl.ds(g * 16, 16)]
      for e in range(16):
        wsp = _bcast16(wv, e)
        r = g * 16 + e
        for j in range(D // 16):
          rows[b][r, pl.ds(j * 16, 16)] = rows[b][r, pl.ds(j * 16, 16)] * wsp
      return 0
    lax.fori_loop(0, CH // 16, grp, 0)

  def _step(i, b):
    pltpu.make_async_copy(hp_hbm.at[pl.ds(0, CH)], rows[b], gsem[b]).wait()
    _scale(i, b)
    pltpu.async_copy(rows[b], acc_sh.at[dstb.at[i]], ssem[b], add=True)
    bn = (b + GLEAD) % NBUF

    @pl.when(jnp.logical_and(i >= NBUF - GLEAD, i + GLEAD < NCPQ))
    def _():
      pltpu.make_async_copy(rows[bn], acc_sh.at[pl.ds(0, CH)],
                            ssem[bn]).wait()

    @pl.when(i + GLEAD < NCPQ)
    def _():
      pltpu.async_copy(hp_hbm.at[srcb.at[i + GLEAD]], rows[bn], gsem[bn])

  def quarter(q, _):
    pltpu.sync_copy(src3_hbm.at[qbase + q], srcb)
    pltpu.sync_copy(dst3_hbm.at[qbase + q], dstb)
    pltpu.sync_copy(ew3_hbm.at[qbase + q], ewb)
    for b in range(GLEAD):
      pltpu.async_copy(hp_hbm.at[srcb.at[b]], rows[b], gsem[b])

    def ring(j, _):
      for t in range(NBUF):
        _step(j * NBUF + t, t)
      return 0
    lax.fori_loop(0, NCPQ // NBUF, ring, 0)
    for b in range(NBUF):
      pltpu.make_async_copy(rows[b], acc_sh.at[pl.ds(0, CH)], ssem[b]).wait()
    return 0
  with jax.named_scope("agg_pipeline"):
    lax.fori_loop(0, nq, quarter, 0)

  with jax.named_scope("agg_tail"):
    plsc.subcore_barrier()
    pltpu.sync_copy(acc_sh.at[pl.ds(s * STR, STR)],
                    out_hbm.at[c, pl.ds(s * STR, STR)])


_agg_call = pl.kernel(
    _agg_body,
    out_type=jax.ShapeDtypeStruct((NC, APAD, D), jnp.float32),
    mesh=plsc.VectorSubcoreMesh(core_axis_name="c", subcore_axis_name="s"),
    scratch_types=[
        pltpu.VMEM((NCPQ, CH), jnp.int32),
        pltpu.VMEM((NCPQ, CH), jnp.int32),
        pltpu.VMEM((NCPQ, CH), jnp.float32),
    ] + [pltpu.VMEM((CH, D), jnp.float32)] * NBUF
      + [pltpu.SemaphoreType.DMA] * (2 * NBUF)
      + [pltpu.VMEM_SHARED((APAD, D), jnp.float32)],
)


def _leaky(x):
  return jnp.where(x >= 0.0, x, 0.1 * x)


def _bn(x, gamma, beta):
  mu = jnp.mean(x, axis=0, keepdims=True)
  xc = x - mu
  var = jnp.mean(xc * xc, axis=0, keepdims=True)
  return gamma * xc * lax.rsqrt(var + 1e-5) + beta


def _tc1_body(x_ref, d0_ref, d1_ref, g_ref, bt_ref, w_ref, h1p_ref, dinv_ref):
  x = x_ref[...]
  xa = _leaky(_bn(x, g_ref[...], bt_ref[...]))
  deg = d0_ref[...] + d1_ref[...] + 1.0
  dinv = jnp.where(deg > 0.0, lax.rsqrt(deg), 0.0)
  h = jnp.dot(xa, w_ref[...], preferred_element_type=jnp.float32)
  h1p_ref[...] = dinv * h
  dinv_ref[...] = dinv


def _tc2_body(p0_ref, p1_ref, hp_ref, dinv_ref, b1_ref, g_ref, bt_ref, w_ref,
              h2p_ref):
  dinv = dinv_ref[...]
  o1 = dinv * (p0_ref[...] + p1_ref[...] + hp_ref[...]) + b1_ref[...]
  a = _leaky(_bn(_leaky(o1), g_ref[...], bt_ref[...]))
  h2 = jnp.dot(a, w_ref[...], preferred_element_type=jnp.float32)
  h2p_ref[...] = dinv * h2


def _tc3_body(q0_ref, q1_ref, hp_ref, dinv_ref, b2_ref, out_ref):
  o = dinv_ref[...] * (q0_ref[...] + q1_ref[...] + hp_ref[...]) + b2_ref[...]
  out_ref[...] = _leaky(o)


_f32 = jnp.float32
_tc1_call = pl.pallas_call(
    _tc1_body,
    out_shape=(jax.ShapeDtypeStruct((N, D), _f32),
               jax.ShapeDtypeStruct((N, 1), _f32)),
)
_tc2_call = pl.pallas_call(
    _tc2_body,
    out_shape=jax.ShapeDtypeStruct((N, D), _f32),
)
_tc3_call = pl.pallas_call(
    _tc3_body,
    out_shape=jax.ShapeDtypeStruct((N, D), _f32),
)


def kernel(x, edge_index, edge_attr, bn1_gamma, bn1_beta, W1, b1,
           bn2_gamma, bn2_beta, W2, b2):
  src = edge_index[0]
  dst = edge_index[1]
  ew = edge_attr[:, 0]

  pad = E_PAD - E
  pi = jnp.arange(pad, dtype=jnp.int32) % N
  src3 = jnp.concatenate([src, pi]).reshape(NQTOT, NCPQ, CH)
  dst3 = jnp.concatenate([dst, pi]).reshape(NQTOT, NCPQ, CH)
  ew3 = jnp.concatenate([ew, jnp.zeros((pad,), jnp.float32)]
                        ).reshape(NQTOT, NCPQ, CH)

  deg_parts = _deg_call(dst3, ew3)
  d0 = deg_parts[:N, None]
  d1 = deg_parts[APAD:APAD + N, None]

  h1p, dinv = _tc1_call(x, d0, d1, bn1_gamma[None, :], bn1_beta[None, :], W1)

  p = _agg_call(h1p, src3, dst3, ew3)
  h2p = _tc2_call(p[0, :N], p[1, :N], h1p, dinv, b1[None, :],
                  bn2_gamma[None, :], bn2_beta[None, :], W2)

  q = _agg_call(h2p, src3, dst3, ew3)
  out = _tc3_call(q[0, :N], q[1, :N], h2p, dinv, b2[None, :])
  return (out, edge_index)


# speedup vs baseline: 1.5111x; 1.0342x over previous
import jax
import jax.numpy as jnp
from jax import lax
from jax.experimental import pallas as pl
from jax.experimental.pallas import tpu as pltpu
from jax.experimental.pallas import tpu_sc as plsc

N = 10000
E = 320000
D = 128
NC, NS = 2, 16
NW = NC * NS
CH = 64
NCPQ = 40
EPQ = NCPQ * CH
NQ0, NQ1 = 4, 4
NQTOT = NS * (NQ0 + NQ1)
E_PAD = NQTOT * EPQ
APAD = 10240
STR = APAD // NS

_GDN = lax.GatherDimensionNumbers(
    offset_dims=(), collapsed_slice_dims=(0,), start_index_map=(0,))


def _bcast16(v, lane):
  idx = jnp.full((16,), lane, jnp.int32)
  return lax.gather(v, idx[:, None], _GDN, (1,),
                    mode=lax.GatherScatterMode.PROMISE_IN_BOUNDS)


def _zero_vmem_1d(ref, n):
  def body(i, _):
    ref[pl.ds(i * 16, 16)] = jnp.zeros((16,), jnp.float32)
    return 0
  lax.fori_loop(0, n // 16, body, 0)


def _zero_vmem_rows(ref, rows):
  def body(i, _):
    for j in range(D // 16):
      ref[i, pl.ds(j * 16, 16)] = jnp.zeros((16,), jnp.float32)
    return 0
  lax.fori_loop(0, rows, body, 0)


def _worker_quarters(c, s):
  qbase = jnp.where(c == 0, s * NQ0, NS * NQ0 + s * NQ1)
  nq = jnp.where(c == 0, NQ0, NQ1)
  return qbase, nq


def _deg_body(dst3_hbm, ew3_hbm, out_hbm, dstb, ewb, z_v, acc_sh, dsem):
  c = lax.axis_index("c")
  s = lax.axis_index("s")
  qbase, nq = _worker_quarters(c, s)

  _zero_vmem_1d(z_v, STR)
  pltpu.sync_copy(z_v, acc_sh.at[pl.ds(s * STR, STR)])
  plsc.subcore_barrier()

  def quarter(q, _):
    pltpu.sync_copy(dst3_hbm.at[qbase + q], dstb)
    pltpu.sync_copy(ew3_hbm.at[qbase + q], ewb)

    def chunk(i, _):
      pltpu.async_copy(ewb.at[i], acc_sh.at[dstb.at[i]], dsem, add=True)

      @pl.when(i >= 16)
      def _():
        pltpu.make_async_copy(ewb.at[0], acc_sh.at[pl.ds(0, CH)],
                              dsem).wait()
      return 0
    lax.fori_loop(0, NCPQ, chunk, 0)
    for _ in range(16):
      pltpu.make_async_copy(ewb.at[0], acc_sh.at[pl.ds(0, CH)],
                            dsem).wait()
    return 0
  lax.fori_loop(0, nq, quarter, 0)

  plsc.subcore_barrier()
  pltpu.sync_copy(acc_sh.at[pl.ds(s * STR, STR)],
                  out_hbm.at[pl.ds(c * APAD + s * STR, STR)])


_deg_call = pl.kernel(
    _deg_body,
    out_type=jax.ShapeDtypeStruct((NC * APAD,), jnp.float32),
    mesh=plsc.VectorSubcoreMesh(core_axis_name="c", subcore_axis_name="s"),
    scratch_types=[
        pltpu.VMEM((NCPQ, CH), jnp.int32),
        pltpu.VMEM((NCPQ, CH), jnp.float32),
        pltpu.VMEM((STR,), jnp.float32),
        pltpu.VMEM_SHARED((APAD,), jnp.float32),
        pltpu.SemaphoreType.DMA,
    ],
)


NBUF = 4
GLEAD = 2


def _agg_body(hp_hbm, src3_hbm, dst3_hbm, ew3_hbm, out_hbm,
              srcb, dstb, ewb, r0, r1, r2, r3,
              g0, g1, g2, g3, s0, s1, s2, s3, acc_sh):
  c = lax.axis_index("c")
  s = lax.axis_index("s")
  qbase, nq = _worker_quarters(c, s)
  rows = (r0, r1, r2, r3)
  gsem = (g0, g1, g2, g3)
  ssem = (s0, s1, s2, s3)

  with jax.named_scope("agg_zero"):
    _zero_vmem_rows(r0, CH)
    for t in range(STR // CH):
      pltpu.sync_copy(r0, acc_sh.at[pl.ds(s * STR + t * CH, CH)])
    plsc.subcore_barrier()

  def _scale(i, b):
    def grp(g, _):
      wv = ewb[i, pl.ds(g * 16, 16)]
      for e in range(16):
        wsp = _bcast16(wv, e)
        r = g * 16 + e
        for j in range(D // 16):
          rows[b][r, pl.ds(j * 16, 16)] = rows[b][r, pl.ds(j * 16, 16)] * wsp
      return 0
    lax.fori_loop(0, CH // 16, grp, 0)

  def _step(i, b):
    pltpu.make_async_copy(hp_hbm.at[pl.ds(0, CH)], rows[b], gsem[b]).wait()
    _scale(i, b)
    pltpu.async_copy(rows[b], acc_sh.at[dstb.at[i]], ssem[b], add=True)
    bn = (b + GLEAD) % NBUF

    @pl.when(jnp.logical_and(i >= NBUF - GLEAD, i + GLEAD < NCPQ))
    def _():
      pltpu.make_async_copy(rows[bn], acc_sh.at[pl.ds(0, CH)],
                            ssem[bn]).wait()

    @pl.when(i + GLEAD < NCPQ)
    def _():
      pltpu.async_copy(hp_hbm.at[srcb.at[i + GLEAD]], rows[bn], gsem[bn])

  def quarter(q, _):
    pltpu.sync_copy(src3_hbm.at[qbase + q], srcb)
    pltpu.sync_copy(dst3_hbm.at[qbase + q], dstb)
    pltpu.sync_copy(ew3_hbm.at[qbase + q], ewb)
    for b in range(GLEAD):
      pltpu.async_copy(hp_hbm.at[srcb.at[b]], rows[b], gsem[b])

    def ring(j, _):
      for t in range(NBUF):
        _step(j * NBUF + t, t)
      return 0
    lax.fori_loop(0, NCPQ // NBUF, ring, 0)
    for b in range(NBUF):
      pltpu.make_async_copy(rows[b], acc_sh.at[pl.ds(0, CH)], ssem[b]).wait()
    return 0
  with jax.named_scope("agg_pipeline"):
    lax.fori_loop(0, nq, quarter, 0)

  with jax.named_scope("agg_tail"):
    plsc.subcore_barrier()
    pltpu.sync_copy(acc_sh.at[pl.ds(s * STR, STR)],
                    out_hbm.at[c, pl.ds(s * STR, STR)])


_agg_call = pl.kernel(
    _agg_body,
    out_type=jax.ShapeDtypeStruct((NC, APAD, D), jnp.float32),
    mesh=plsc.VectorSubcoreMesh(core_axis_name="c", subcore_axis_name="s"),
    scratch_types=[
        pltpu.VMEM((NCPQ, CH), jnp.int32),
        pltpu.VMEM((NCPQ, CH), jnp.int32),
        pltpu.VMEM((NCPQ, CH), jnp.float32),
    ] + [pltpu.VMEM((CH, D), jnp.float32)] * NBUF
      + [pltpu.SemaphoreType.DMA] * (2 * NBUF)
      + [pltpu.VMEM_SHARED((APAD, D), jnp.float32)],
)


def _leaky(x):
  return jnp.where(x >= 0.0, x, 0.1 * x)


def _bn(x, gamma, beta):
  mu = jnp.mean(x, axis=0, keepdims=True)
  xc = x - mu
  var = jnp.mean(xc * xc, axis=0, keepdims=True)
  return gamma * xc * lax.rsqrt(var + 1e-5) + beta


def _tc1a_body(x_ref, g_ref, bt_ref, w_ref, h1_ref):
  xa = _leaky(_bn(x_ref[...], g_ref[...], bt_ref[...]))
  h1_ref[...] = jnp.dot(xa, w_ref[...], preferred_element_type=jnp.float32)


def _tc1b_body(h_ref, d0_ref, d1_ref, h1p_ref, dinv_ref):
  deg = d0_ref[...] + d1_ref[...] + 1.0
  dinv = jnp.where(deg > 0.0, lax.rsqrt(deg), 0.0)
  h1p_ref[...] = dinv * h_ref[...]
  dinv_ref[...] = dinv


def _tc2_body(p_ref, hp_ref, dinv_ref, b1_ref, g_ref, bt_ref, w_ref,
              h2p_ref):
  dinv = dinv_ref[...]
  agg = p_ref[0, :N, :] + p_ref[1, :N, :]
  o1 = dinv * (agg + hp_ref[...]) + b1_ref[...]
  a = _leaky(_bn(_leaky(o1), g_ref[...], bt_ref[...]))
  h2 = jnp.dot(a, w_ref[...], preferred_element_type=jnp.float32)
  h2p_ref[...] = dinv * h2


def _tc3_body(q_ref, hp_ref, dinv_ref, b2_ref, out_ref):
  agg = q_ref[0, :N, :] + q_ref[1, :N, :]
  o = dinv_ref[...] * (agg + hp_ref[...]) + b2_ref[...]
  out_ref[...] = _leaky(o)


_f32 = jnp.float32
_tc1a_call = pl.pallas_call(
    _tc1a_body,
    out_shape=jax.ShapeDtypeStruct((N, D), _f32),
)
_tc1b_call = pl.pallas_call(
    _tc1b_body,
    out_shape=(jax.ShapeDtypeStruct((N, D), _f32),
               jax.ShapeDtypeStruct((N, 1), _f32)),
)
_tc2_call = pl.pallas_call(
    _tc2_body,
    out_shape=jax.ShapeDtypeStruct((N, D), _f32),
)
_tc3_call = pl.pallas_call(
    _tc3_body,
    out_shape=jax.ShapeDtypeStruct((N, D), _f32),
)


def kernel(x, edge_index, edge_attr, bn1_gamma, bn1_beta, W1, b1,
           bn2_gamma, bn2_beta, W2, b2):
  src = edge_index[0]
  dst = edge_index[1]
  ew = edge_attr[:, 0]

  pad = E_PAD - E
  pi = jnp.arange(pad, dtype=jnp.int32) % N
  src3 = jnp.concatenate([src, pi]).reshape(NQTOT, NCPQ, CH)
  dst3 = jnp.concatenate([dst, pi]).reshape(NQTOT, NCPQ, CH)
  ew3 = jnp.concatenate([ew, jnp.zeros((pad,), jnp.float32)]
                        ).reshape(NQTOT, NCPQ, CH)

  deg_parts = _deg_call(dst3, ew3)
  h1 = _tc1a_call(x, bn1_gamma[None, :], bn1_beta[None, :], W1)
  d0 = deg_parts[:N, None]
  d1 = deg_parts[APAD:APAD + N, None]
  h1p, dinv = _tc1b_call(h1, d0, d1)

  p = _agg_call(h1p, src3, dst3, ew3)
  h2p = _tc2_call(p, h1p, dinv, b1[None, :],
                  bn2_gamma[None, :], bn2_beta[None, :], W2)

  q = _agg_call(h2p, src3, dst3, ew3)
  out = _tc3_call(q, h2p, dinv, b2[None, :])
  return (out, edge_index)


# speedup vs baseline: 1.5226x; 1.0076x over previous
import jax
import jax.numpy as jnp
from jax import lax
from jax.experimental import pallas as pl
from jax.experimental.pallas import tpu as pltpu
from jax.experimental.pallas import tpu_sc as plsc

N = 10000
E = 320000
D = 128
NC, NS = 2, 16
NW = NC * NS
CH = 64
NCPQ = 40
EPQ = NCPQ * CH
NQ0, NQ1 = 4, 4
NQTOT = NS * (NQ0 + NQ1)
E_PAD = NQTOT * EPQ
APAD = 10240
STR = APAD // NS

_GDN = lax.GatherDimensionNumbers(
    offset_dims=(), collapsed_slice_dims=(0,), start_index_map=(0,))


def _bcast16(v, lane):
  idx = jnp.full((16,), lane, jnp.int32)
  return lax.gather(v, idx[:, None], _GDN, (1,),
                    mode=lax.GatherScatterMode.PROMISE_IN_BOUNDS)


def _zero_vmem_1d(ref, n):
  def body(i, _):
    ref[pl.ds(i * 16, 16)] = jnp.zeros((16,), jnp.float32)
    return 0
  lax.fori_loop(0, n // 16, body, 0)


def _zero_vmem_rows(ref, rows):
  def body(i, _):
    for j in range(D // 16):
      ref[i, pl.ds(j * 16, 16)] = jnp.zeros((16,), jnp.float32)
    return 0
  lax.fori_loop(0, rows, body, 0)


def _worker_quarters(c, s):
  qbase = jnp.where(c == 0, s * NQ0, NS * NQ0 + s * NQ1)
  nq = jnp.where(c == 0, NQ0, NQ1)
  return qbase, nq


def _deg_body(dst3_hbm, ew1_hbm, out_hbm, dstb, ewb, z_v, acc_sh, dsem):
  c = lax.axis_index("c")
  s = lax.axis_index("s")
  qbase, nq = _worker_quarters(c, s)

  _zero_vmem_1d(z_v, STR)
  pltpu.sync_copy(z_v, acc_sh.at[pl.ds(s * STR, STR)])
  plsc.subcore_barrier()

  def quarter(q, _):
    pltpu.sync_copy(dst3_hbm.at[qbase + q], dstb)
    pltpu.sync_copy(ew1_hbm.at[pl.ds((qbase + q) * EPQ, EPQ)], ewb)

    def chunk(i, _):
      pltpu.async_copy(ewb.at[pl.ds(i * CH, CH)], acc_sh.at[dstb.at[i]],
                       dsem, add=True)

      @pl.when(i >= 16)
      def _():
        pltpu.make_async_copy(ewb.at[pl.ds(0, CH)], acc_sh.at[pl.ds(0, CH)],
                              dsem).wait()
      return 0
    lax.fori_loop(0, NCPQ, chunk, 0)
    for _ in range(16):
      pltpu.make_async_copy(ewb.at[pl.ds(0, CH)], acc_sh.at[pl.ds(0, CH)],
                            dsem).wait()
    return 0
  lax.fori_loop(0, nq, quarter, 0)

  plsc.subcore_barrier()
  pltpu.sync_copy(acc_sh.at[pl.ds(s * STR, STR)],
                  out_hbm.at[pl.ds(c * APAD + s * STR, STR)])


_deg_call = pl.kernel(
    _deg_body,
    out_type=jax.ShapeDtypeStruct((NC * APAD,), jnp.float32),
    mesh=plsc.VectorSubcoreMesh(core_axis_name="c", subcore_axis_name="s"),
    scratch_types=[
        pltpu.VMEM((NCPQ, CH), jnp.int32),
        pltpu.VMEM((EPQ,), jnp.float32),
        pltpu.VMEM((STR,), jnp.float32),
        pltpu.VMEM_SHARED((APAD,), jnp.float32),
        pltpu.SemaphoreType.DMA,
    ],
)


NBUF = 4
GLEAD = 2


def _agg_body(hp_hbm, src1_hbm, dst3_hbm, ew1_hbm, out_hbm,
              srcb, dstb, ewb, r0, r1, r2, r3,
              g0, g1, g2, g3, s0, s1, s2, s3, acc_sh):
  c = lax.axis_index("c")
  s = lax.axis_index("s")
  qbase, nq = _worker_quarters(c, s)
  rows = (r0, r1, r2, r3)
  gsem = (g0, g1, g2, g3)
  ssem = (s0, s1, s2, s3)

  with jax.named_scope("agg_zero"):
    _zero_vmem_rows(r0, CH)
    for t in range(STR // CH):
      pltpu.sync_copy(r0, acc_sh.at[pl.ds(s * STR + t * CH, CH)])
    plsc.subcore_barrier()

  def _scale(i, b):
    def grp(g, _):
      wv = ewb[pl.ds(i * CH + g * 16, 16)]
      for e in range(16):
        wsp = _bcast16(wv, e)
        r = g * 16 + e
        for j in range(D // 16):
          rows[b][r, pl.ds(j * 16, 16)] = rows[b][r, pl.ds(j * 16, 16)] * wsp
      return 0
    lax.fori_loop(0, CH // 16, grp, 0)

  def _step(i, b):
    pltpu.make_async_copy(hp_hbm.at[pl.ds(0, CH)], rows[b], gsem[b]).wait()
    _scale(i, b)
    pltpu.async_copy(rows[b], acc_sh.at[dstb.at[i]], ssem[b], add=True)
    bn = (b + GLEAD) % NBUF

    @pl.when(jnp.logical_and(i >= NBUF - GLEAD, i + GLEAD < NCPQ))
    def _():
      pltpu.make_async_copy(rows[bn], acc_sh.at[pl.ds(0, CH)],
                            ssem[bn]).wait()

    @pl.when(i + GLEAD < NCPQ)
    def _():
      pltpu.async_copy(hp_hbm.at[srcb.at[pl.ds((i + GLEAD) * CH, CH)]],
                       rows[bn], gsem[bn])

  def quarter(q, _):
    pltpu.sync_copy(src1_hbm.at[pl.ds((qbase + q) * EPQ, EPQ)], srcb)
    pltpu.sync_copy(dst3_hbm.at[qbase + q], dstb)
    pltpu.sync_copy(ew1_hbm.at[pl.ds((qbase + q) * EPQ, EPQ)], ewb)
    for b in range(GLEAD):
      pltpu.async_copy(hp_hbm.at[srcb.at[pl.ds(b * CH, CH)]], rows[b],
                       gsem[b])

    def ring(j, _):
      for t in range(NBUF):
        _step(j * NBUF + t, t)
      return 0
    lax.fori_loop(0, NCPQ // NBUF, ring, 0)
    for b in range(NBUF):
      pltpu.make_async_copy(rows[b], acc_sh.at[pl.ds(0, CH)], ssem[b]).wait()
    return 0
  with jax.named_scope("agg_pipeline"):
    lax.fori_loop(0, nq, quarter, 0)

  with jax.named_scope("agg_tail"):
    plsc.subcore_barrier()
    pltpu.sync_copy(acc_sh.at[pl.ds(s * STR, STR)],
                    out_hbm.at[c, pl.ds(s * STR, STR)])


_agg_call = pl.kernel(
    _agg_body,
    out_type=jax.ShapeDtypeStruct((NC, APAD, D), jnp.float32),
    mesh=plsc.VectorSubcoreMesh(core_axis_name="c", subcore_axis_name="s"),
    scratch_types=[
        pltpu.VMEM((EPQ,), jnp.int32),
        pltpu.VMEM((NCPQ, CH), jnp.int32),
        pltpu.VMEM((EPQ,), jnp.float32),
    ] + [pltpu.VMEM((CH, D), jnp.float32)] * NBUF
      + [pltpu.SemaphoreType.DMA] * (2 * NBUF)
      + [pltpu.VMEM_SHARED((APAD, D), jnp.float32)],
)


def _leaky(x):
  return jnp.where(x >= 0.0, x, 0.1 * x)


def _bn(x, gamma, beta):
  mu = jnp.mean(x, axis=0, keepdims=True)
  xc = x - mu
  var = jnp.mean(xc * xc, axis=0, keepdims=True)
  return gamma * xc * lax.rsqrt(var + 1e-5) + beta


def _tc1a_body(x_ref, g_ref, bt_ref, w_ref, h1_ref):
  xa = _leaky(_bn(x_ref[...], g_ref[...], bt_ref[...]))
  h1_ref[...] = jnp.dot(xa, w_ref[...], preferred_element_type=jnp.float32)


def _tc1b_body(h_ref, d0_ref, d1_ref, h1p_ref, dinv_ref):
  deg = d0_ref[...] + d1_ref[...] + 1.0
  dinv = jnp.where(deg > 0.0, lax.rsqrt(deg), 0.0)
  h1p_ref[...] = dinv * h_ref[...]
  dinv_ref[...] = dinv


def _tc2_body(p_ref, hp_ref, dinv_ref, b1_ref, g_ref, bt_ref, w_ref,
              h2p_ref):
  dinv = dinv_ref[...]
  agg = p_ref[0, :N, :] + p_ref[1, :N, :]
  o1 = dinv * (agg + hp_ref[...]) + b1_ref[...]
  a = _leaky(_bn(_leaky(o1), g_ref[...], bt_ref[...]))
  h2 = jnp.dot(a, w_ref[...], preferred_element_type=jnp.float32)
  h2p_ref[...] = dinv * h2


def _tc3_body(q_ref, hp_ref, dinv_ref, b2_ref, out_ref):
  agg = q_ref[0, :N, :] + q_ref[1, :N, :]
  o = dinv_ref[...] * (agg + hp_ref[...]) + b2_ref[...]
  out_ref[...] = _leaky(o)


_f32 = jnp.float32
_tc1a_call = pl.pallas_call(
    _tc1a_body,
    out_shape=jax.ShapeDtypeStruct((N, D), _f32),
)
_tc1b_call = pl.pallas_call(
    _tc1b_body,
    out_shape=(jax.ShapeDtypeStruct((N, D), _f32),
               jax.ShapeDtypeStruct((N, 1), _f32)),
)
_tc2_call = pl.pallas_call(
    _tc2_body,
    out_shape=jax.ShapeDtypeStruct((N, D), _f32),
)
_tc3_call = pl.pallas_call(
    _tc3_body,
    out_shape=jax.ShapeDtypeStruct((N, D), _f32),
)


def kernel(x, edge_index, edge_attr, bn1_gamma, bn1_beta, W1, b1,
           bn2_gamma, bn2_beta, W2, b2):
  src = edge_index[0]
  dst = edge_index[1]
  ew = edge_attr[:, 0]

  pad = E_PAD - E
  pi = jnp.arange(pad, dtype=jnp.int32) % N
  src1 = jnp.concatenate([src, pi])
  dst3 = jnp.concatenate([dst, pi]).reshape(NQTOT, NCPQ, CH)
  ew1 = jnp.concatenate([ew, jnp.zeros((pad,), jnp.float32)])

  deg_parts = _deg_call(dst3, ew1)
  h1 = _tc1a_call(x, bn1_gamma[None, :], bn1_beta[None, :], W1)
  d0 = deg_parts[:N, None]
  d1 = deg_parts[APAD:APAD + N, None]
  h1p, dinv = _tc1b_call(h1, d0, d1)

  p = _agg_call(h1p, src1, dst3, ew1)
  h2p = _tc2_call(p, h1p, dinv, b1[None, :],
                  bn2_gamma[None, :], bn2_beta[None, :], W2)

  q = _agg_call(h2p, src1, dst3, ew1)
  out = _tc3_call(q, h2p, dinv, b2[None, :])
  return (out, edge_index)
